# baseline (speedup 1.0000x reference)
import functools

import jax
import jax.numpy as jnp
from jax import lax
from jax.experimental import pallas as pl
from jax.experimental.pallas import tpu as pltpu, tpu_sc as plsc

_NC = 2
_NS = 16
_NW = _NC * _NS

_CHUNK = 512
_NBUF = 2


def _make_gather(B, D):
  assert B % (_NW * _CHUNK) == 0
  b_per_w = B // _NW
  n_chunks = b_per_w // _CHUNK
  mesh = plsc.VectorSubcoreMesh(core_axis_name="c", subcore_axis_name="s")

  @functools.partial(
      pl.kernel,
      mesh=mesh,
      out_type=jax.ShapeDtypeStruct((B, D), jnp.float32),
      scratch_types=[
          pltpu.VMEM((_NBUF, _CHUNK), jnp.int32),
          pltpu.VMEM((_NBUF, _CHUNK, D), jnp.float32),
          pltpu.SemaphoreType.DMA,
          pltpu.SemaphoreType.DMA,
          pltpu.SemaphoreType.DMA,
      ],
      compiler_params=pltpu.CompilerParams(use_tc_tiling_on_sc=False),
  )
  def gather_kernel(table_hbm, idx_hbm, out_hbm, idx_v, rows_v, isem, gsem,
                    ssem):
    wid = lax.axis_index("s") * _NC + lax.axis_index("c")
    base = wid * b_per_w

    def start_idx_load(chunk, slot):
      pltpu.async_copy(idx_hbm.at[pl.ds(base + chunk * _CHUNK, _CHUNK)],
                       idx_v.at[slot], isem)

    def wait_idx_load(slot):
      pltpu.make_async_copy(idx_hbm.at[pl.ds(base, _CHUNK)], idx_v.at[slot],
                            isem).wait()

    def start_gather(slot):
      pltpu.async_copy(table_hbm.at[idx_v.at[slot]], rows_v.at[slot], gsem)

    def wait_gather(slot):
      pltpu.make_async_copy(table_hbm.at[idx_v.at[slot]], rows_v.at[slot],
                            gsem).wait()

    def start_store(chunk, slot):
      pltpu.async_copy(rows_v.at[slot],
                       out_hbm.at[pl.ds(base + chunk * _CHUNK, _CHUNK)], ssem)

    def wait_store(slot):
      pltpu.make_async_copy(rows_v.at[slot],
                            out_hbm.at[pl.ds(base, _CHUNK)], ssem).wait()

    for slot in range(_NBUF):
      start_idx_load(slot, slot)

    def step(g, carry):
      slot = lax.rem(g, _NBUF)

      def do(fn, *a):
        return lambda: fn(*a)

      wait_idx_load(slot)
      pl.when(g >= _NBUF)(do(wait_store, slot))
      start_gather(slot)
      wait_gather(slot)
      start_store(g, slot)
      pl.when(g + _NBUF < n_chunks)(do(start_idx_load, g + _NBUF, slot))
      return carry

    lax.fori_loop(0, n_chunks, step, 0)

    for slot in range(min(_NBUF, n_chunks)):
      wait_store(slot)

  return gather_kernel


@jax.jit
def kernel(input_ids, table):
  B = input_ids.shape[0] * input_ids.shape[1]
  V, D = table.shape
  flat_idx = input_ids.reshape(B).astype(jnp.int32)
  out = _make_gather(B, D)(table, flat_idx)
  return out.reshape(input_ids.shape[0], input_ids.shape[1], D)


# speedup vs baseline: 1.0018x; 1.0018x over previous
import functools

import jax
import jax.numpy as jnp
from jax import lax
from jax.experimental import pallas as pl
from jax.experimental.pallas import tpu as pltpu, tpu_sc as plsc

_NC = 2
_NS = 16
_NW = _NC * _NS

_CHUNK = 800
_NBUF = 2


def _make_gather(B, D):
  assert B % (_NW * _CHUNK) == 0
  b_per_w = B // _NW
  n_chunks = b_per_w // _CHUNK
  mesh = plsc.VectorSubcoreMesh(core_axis_name="c", subcore_axis_name="s")

  @functools.partial(
      pl.kernel,
      mesh=mesh,
      out_type=jax.ShapeDtypeStruct((B, D), jnp.float32),
      scratch_types=[
          pltpu.VMEM((_NBUF, _CHUNK), jnp.int32),
          pltpu.VMEM((_NBUF, _CHUNK, D), jnp.float32),
          pltpu.SemaphoreType.DMA,
          pltpu.SemaphoreType.DMA,
          pltpu.SemaphoreType.DMA,
      ],
      compiler_params=pltpu.CompilerParams(use_tc_tiling_on_sc=False),
  )
  def gather_kernel(table_hbm, idx_hbm, out_hbm, idx_v, rows_v, isem, gsem,
                    ssem):
    wid = lax.axis_index("s") * _NC + lax.axis_index("c")
    base = wid * b_per_w

    def start_idx_load(chunk, slot):
      pltpu.async_copy(idx_hbm.at[pl.ds(base + chunk * _CHUNK, _CHUNK)],
                       idx_v.at[slot], isem)

    def wait_idx_load(slot):
      pltpu.make_async_copy(idx_hbm.at[pl.ds(base, _CHUNK)], idx_v.at[slot],
                            isem).wait()

    def start_gather(slot):
      pltpu.async_copy(table_hbm.at[idx_v.at[slot]], rows_v.at[slot], gsem)

    def wait_gather(slot):
      pltpu.make_async_copy(table_hbm.at[idx_v.at[slot]], rows_v.at[slot],
                            gsem).wait()

    def start_store(chunk, slot):
      pltpu.async_copy(rows_v.at[slot],
                       out_hbm.at[pl.ds(base + chunk * _CHUNK, _CHUNK)], ssem)

    def wait_store(slot):
      pltpu.make_async_copy(rows_v.at[slot],
                            out_hbm.at[pl.ds(base, _CHUNK)], ssem).wait()

    for slot in range(_NBUF):
      start_idx_load(slot, slot)

    def step(g, carry):
      slot = lax.rem(g, _NBUF)

      def do(fn, *a):
        return lambda: fn(*a)

      wait_idx_load(slot)
      pl.when(g >= _NBUF)(do(wait_store, slot))
      start_gather(slot)
      wait_gather(slot)
      start_store(g, slot)
      pl.when(g + _NBUF < n_chunks)(do(start_idx_load, g + _NBUF, slot))
      return carry

    lax.fori_loop(0, n_chunks, step, 0)

    for slot in range(min(_NBUF, n_chunks)):
      wait_store(slot)

  return gather_kernel


@jax.jit
def kernel(input_ids, table):
  B = input_ids.shape[0] * input_ids.shape[1]
  V, D = table.shape
  flat_idx = input_ids.reshape(B).astype(jnp.int32)
  out = _make_gather(B, D)(table, flat_idx)
  return out.reshape(input_ids.shape[0], input_ids.shape[1], D)


# speedup vs baseline: 1.0023x; 1.0005x over previous
import functools

import jax
import jax.numpy as jnp
from jax import lax
from jax.experimental import pallas as pl
from jax.experimental.pallas import tpu as pltpu, tpu_sc as plsc

_NC = 2
_NS = 16
_NW = _NC * _NS

_CHUNK = 800
_NBUF = 2


def _make_gather(B, D):
  assert B % (_NW * _CHUNK) == 0
  b_per_w = B // _NW
  n_chunks = b_per_w // _CHUNK
  mesh = plsc.VectorSubcoreMesh(core_axis_name="c", subcore_axis_name="s")

  @functools.partial(
      pl.kernel,
      mesh=mesh,
      out_type=jax.ShapeDtypeStruct((B, D), jnp.float32),
      scratch_types=[
          pltpu.VMEM((_NBUF, _CHUNK), jnp.int32),
          pltpu.VMEM((_NBUF, _CHUNK, D), jnp.float32),
          pltpu.SemaphoreType.DMA((_NBUF,)),
          pltpu.SemaphoreType.DMA((_NBUF,)),
          pltpu.SemaphoreType.DMA((_NBUF,)),
      ],
      compiler_params=pltpu.CompilerParams(use_tc_tiling_on_sc=False),
  )
  def gather_kernel(table_hbm, idx_hbm, out_hbm, idx_v, rows_v, isem, gsem,
                    ssem):
    wid = lax.axis_index("s") * _NC + lax.axis_index("c")
    base = wid * b_per_w

    def start_idx_load(chunk, slot):
      pltpu.async_copy(idx_hbm.at[pl.ds(base + chunk * _CHUNK, _CHUNK)],
                       idx_v.at[slot], isem.at[slot])

    def wait_idx_load(slot):
      pltpu.make_async_copy(idx_hbm.at[pl.ds(base, _CHUNK)], idx_v.at[slot],
                            isem.at[slot]).wait()

    def start_gather(slot):
      pltpu.async_copy(table_hbm.at[idx_v.at[slot]], rows_v.at[slot],
                       gsem.at[slot])

    def wait_gather(slot):
      pltpu.make_async_copy(table_hbm.at[idx_v.at[slot]], rows_v.at[slot],
                            gsem.at[slot]).wait()

    def start_store(chunk, slot):
      pltpu.async_copy(rows_v.at[slot],
                       out_hbm.at[pl.ds(base + chunk * _CHUNK, _CHUNK)],
                       ssem.at[slot])

    def wait_store(slot):
      pltpu.make_async_copy(rows_v.at[slot],
                            out_hbm.at[pl.ds(base, _CHUNK)],
                            ssem.at[slot]).wait()

    for slot in range(_NBUF):
      start_idx_load(slot, slot)

    def step(g, carry):
      slot = lax.rem(g, _NBUF)
      prev = lax.rem(g + 1, _NBUF)

      def do(fn, *a):
        return lambda: fn(*a)

      wait_idx_load(slot)
      pl.when(g >= _NBUF)(do(wait_store, slot))
      start_gather(slot)

      def drain_prev():
        wait_gather(prev)
        start_store(g - 1, prev)
        pl.when(g + 1 < n_chunks)(do(start_idx_load, g + 1, prev))

      pl.when(g >= 1)(drain_prev)
      return carry

    lax.fori_loop(0, n_chunks, step, 0)

    last = n_chunks - 1
    lslot = last % _NBUF
    wait_gather(lslot)
    start_store(last, lslot)
    for slot in range(min(_NBUF, n_chunks)):
      wait_store(slot)

  return gather_kernel


@jax.jit
def kernel(input_ids, table):
  B = input_ids.shape[0] * input_ids.shape[1]
  V, D = table.shape
  flat_idx = input_ids.reshape(B).astype(jnp.int32)
  out = _make_gather(B, D)(table, flat_idx)
  return out.reshape(input_ids.shape[0], input_ids.shape[1], D)


# speedup vs baseline: 1.0229x; 1.0206x over previous
import functools

import jax
import jax.numpy as jnp
from jax import lax
from jax.experimental import pallas as pl
from jax.experimental.pallas import tpu as pltpu, tpu_sc as plsc

_NC = 2
_NS = 16
_NW = _NC * _NS

_CHUNK = 800
_NBUF = 2


def _make_gather(B, D):
  assert B % (_NW * _CHUNK) == 0
  b_per_w = B // _NW
  n_chunks = b_per_w // _CHUNK
  mesh = plsc.VectorSubcoreMesh(core_axis_name="c", subcore_axis_name="s")

  @functools.partial(
      pl.kernel,
      mesh=mesh,
      out_type=jax.ShapeDtypeStruct((B, D), jnp.float32),
      scratch_types=[
          pltpu.VMEM((_NBUF, _CHUNK), jnp.int32),
          pltpu.VMEM((_NBUF, _CHUNK, D), jnp.float32),
          pltpu.SemaphoreType.DMA((_NBUF,)),
          pltpu.SemaphoreType.DMA((_NBUF,)),
          pltpu.SemaphoreType.DMA((_NBUF,)),
      ],
      compiler_params=pltpu.CompilerParams(use_tc_tiling_on_sc=False),
  )
  def gather_kernel(table_hbm, idx_hbm, out_hbm, idx_v, rows_v, isem, gsem,
                    ssem):
    wid = lax.axis_index("s") * _NC + lax.axis_index("c")
    base = wid * b_per_w

    def start_idx_load(chunk, slot):
      pltpu.async_copy(idx_hbm.at[pl.ds(base + chunk * _CHUNK, _CHUNK)],
                       idx_v.at[slot], isem.at[slot])

    def wait_idx_load(slot):
      pltpu.make_async_copy(idx_hbm.at[pl.ds(base, _CHUNK)], idx_v.at[slot],
                            isem.at[slot]).wait()

    def start_gather(slot):
      pltpu.async_copy(table_hbm.at[idx_v.at[slot]], rows_v.at[slot],
                       gsem.at[slot])

    def wait_gather(slot):
      pltpu.make_async_copy(table_hbm.at[idx_v.at[slot]], rows_v.at[slot],
                            gsem.at[slot]).wait()

    def start_store(chunk, slot):
      pltpu.async_copy(rows_v.at[slot],
                       out_hbm.at[pl.ds(base + chunk * _CHUNK, _CHUNK)],
                       ssem.at[slot])

    def wait_store(slot):
      pltpu.make_async_copy(rows_v.at[slot],
                            out_hbm.at[pl.ds(base, _CHUNK)],
                            ssem.at[slot]).wait()

    for slot in range(_NBUF):
      start_idx_load(slot, slot)

    def step(g, carry):
      slot = lax.rem(g, _NBUF)
      prev = lax.rem(g + 1, _NBUF)

      def do(fn, *a):
        return lambda: fn(*a)

      wait_idx_load(slot)
      pl.when(g >= _NBUF)(do(wait_store, slot))
      start_gather(slot)

      def drain_prev():
        wait_gather(prev)
        start_store(g - 1, prev)
        pl.when(g + 1 < n_chunks)(do(start_idx_load, g + 1, prev))

      pl.when(g >= 1)(drain_prev)
      return carry

    lax.fori_loop(0, n_chunks, step, 0)

    last = n_chunks - 1
    lslot = last % _NBUF
    wait_gather(lslot)
    start_store(last, lslot)
    for slot in range(min(_NBUF, n_chunks)):
      wait_store(slot)

  return gather_kernel


@jax.jit
def kernel(input_ids, table):
  B = input_ids.shape[0] * input_ids.shape[1]
  V, D = table.shape
  t2 = jnp.pad(table, ((0, 0), (0, 128 - D))).reshape(2 * V, D)
  flat_idx = input_ids.reshape(B).astype(jnp.int32) * 2
  out = _make_gather(B, D)(t2, flat_idx)
  return out.reshape(input_ids.shape[0], input_ids.shape[1], D)
